# baseline (speedup 1.0000x reference)
import functools

import jax
import jax.numpy as jnp
from jax import lax
from jax.experimental import pallas as pl
from jax.experimental.pallas import tpu as pltpu
from jax.experimental.pallas import tpu_sc as plsc

B = 4096
H = 50
TOTAL = B * H
D = 64
DP = 128
VP = 100008
NC = 2
NS = 16
NW = NC * NS
B_PER_W = B // NW
PER_W = TOTAL // NW
BCH = 4
CHUNK = BCH * H
NCHUNK = PER_W // CHUNK
NBUF = 4


@functools.partial(
    pl.kernel,
    mesh=plsc.VectorSubcoreMesh(core_axis_name="c", subcore_axis_name="s"),
    out_type=jax.ShapeDtypeStruct((B, H, DP), jnp.float32),
    scratch_types=(
        [pltpu.VMEM((CHUNK,), jnp.int32) for _ in range(NBUF)]
        + [pltpu.VMEM((CHUNK, DP), jnp.float32) for _ in range(NBUF)]
        + [pltpu.SemaphoreType.DMA] * (3 * NBUF)
    ),
    compiler_params=pltpu.CompilerParams(
        disable_bounds_checks=True,
        disable_semaphore_checks=True,
    ),
)
def _gather_rows(idx_hbm, table_hbm, out_hbm, *scratch):
    idxs = scratch[:NBUF]
    bufs = scratch[NBUF:2 * NBUF]
    isems = scratch[2 * NBUF:3 * NBUF]
    gsems = scratch[3 * NBUF:4 * NBUF]
    ssems = scratch[4 * NBUF:5 * NBUF]
    wid = lax.axis_index("s") * NC + lax.axis_index("c")
    base = wid * PER_W
    bbase = wid * B_PER_W

    def idxload(g):
        return pltpu.make_async_copy(
            idx_hbm.at[pl.ds(base + g * CHUNK, CHUNK)],
            idxs[g % NBUF], isems[g % NBUF])

    def gather(g):
        return pltpu.make_async_copy(
            table_hbm.at[idxs[g % NBUF]], bufs[g % NBUF], gsems[g % NBUF])

    def stores(g):
        return [
            pltpu.make_async_copy(
                bufs[g % NBUF].at[pl.ds(i * H, H)],
                out_hbm.at[bbase + g * BCH + i],
                ssems[g % NBUF])
            for i in range(BCH)
        ]

    for g in range(NBUF):
        idxload(g).start()
    idxload(0).wait()
    gather(0).start()
    idxload(1).wait()
    gather(1).start()
    for g in range(NCHUNK):
        if g >= 2:
            for s in stores(g - 2):
                s.wait()
        if g + 2 < NCHUNK:
            idxload(g + 2).wait()
            gather(g + 2).start()
        gather(g).wait()
        if g + NBUF < NCHUNK:
            idxload(g + NBUF).start()
        for s in stores(g):
            s.start()
    for s in stores(NCHUNK - 2):
        s.wait()
    for s in stores(NCHUNK - 1):
        s.wait()


def kernel(user_ids, ID_embeddings):
    idx = user_ids.reshape(-1).astype(jnp.int32)
    table = jnp.pad(ID_embeddings,
                    ((0, VP - ID_embeddings.shape[0]), (0, DP - D)))
    out = _gather_rows(idx, table)
    return out[:, :, :D]


# speedup vs baseline: 1.0013x; 1.0013x over previous
import functools

import jax
import jax.numpy as jnp
from jax import lax
from jax.experimental import pallas as pl
from jax.experimental.pallas import tpu as pltpu
from jax.experimental.pallas import tpu_sc as plsc

B = 4096
H = 50
TOTAL = B * H
D = 64
DP = 128
VP = 100008
NC = 2
NS = 16
NW = NC * NS
B_PER_W = B // NW
PER_W = TOTAL // NW
BCH = 4
CHUNK = BCH * H
NCHUNK = PER_W // CHUNK
NBUF = 4


@functools.partial(
    pl.kernel,
    mesh=plsc.VectorSubcoreMesh(core_axis_name="c", subcore_axis_name="s"),
    out_type=jax.ShapeDtypeStruct((B, H, DP), jnp.float32),
    scratch_types=(
        [pltpu.VMEM((CHUNK,), jnp.int32) for _ in range(NBUF)]
        + [pltpu.VMEM((CHUNK, DP), jnp.float32) for _ in range(NBUF)]
        + [pltpu.SemaphoreType.DMA] * (3 * NBUF)
    ),
)
def _gather_rows(idx_hbm, table_hbm, out_hbm, *scratch):
    idxs = scratch[:NBUF]
    bufs = scratch[NBUF:2 * NBUF]
    isems = scratch[2 * NBUF:3 * NBUF]
    gsems = scratch[3 * NBUF:4 * NBUF]
    ssems = scratch[4 * NBUF:5 * NBUF]
    wid = lax.axis_index("s") * NC + lax.axis_index("c")
    base = wid * PER_W
    bbase = wid * B_PER_W

    def idxload(g):
        return pltpu.make_async_copy(
            idx_hbm.at[pl.ds(base + g * CHUNK, CHUNK)],
            idxs[g % NBUF], isems[g % NBUF])

    def gather(g):
        return pltpu.make_async_copy(
            table_hbm.at[idxs[g % NBUF]], bufs[g % NBUF], gsems[g % NBUF])

    def stores(g):
        return [
            pltpu.make_async_copy(
                bufs[g % NBUF].at[pl.ds(i * H, H)],
                out_hbm.at[bbase + g * BCH + i],
                ssems[g % NBUF])
            for i in range(BCH)
        ]

    for g in range(NBUF):
        idxload(g).start()
    idxload(0).wait()
    gather(0).start()
    idxload(1).wait()
    gather(1).start()
    for g in range(NCHUNK):
        if g >= 2:
            for s in stores(g - 2):
                s.wait()
        if g + 2 < NCHUNK:
            idxload(g + 2).wait()
            gather(g + 2).start()
        gather(g).wait()
        if g + NBUF < NCHUNK:
            idxload(g + NBUF).start()
        for s in stores(g):
            s.start()
    for s in stores(NCHUNK - 2):
        s.wait()
    for s in stores(NCHUNK - 1):
        s.wait()


def kernel(user_ids, ID_embeddings):
    idx = user_ids.reshape(-1).astype(jnp.int32)
    table = jnp.pad(ID_embeddings,
                    ((0, VP - ID_embeddings.shape[0]), (0, DP - D)))
    out = _gather_rows(idx, table)
    return out[:, :, :D]
